# baseline (speedup 1.0000x reference)
import functools

import jax
import jax.numpy as jnp
from jax import lax
from jax.experimental import pallas as pl
from jax.experimental.pallas import tpu as pltpu
from jax.experimental.pallas import tpu_sc as plsc

_N = 10000
_D = 128
_DE = 144
_E = 160000
_BLK = 80
_CHUNK = 80
_TILES = 16
_NPAD = 10240
_RPT = _NPAD // _TILES
_EPT = _E // _TILES
_NCHUNK = _EPT // _CHUNK
_IBLK = 2000
_BCHUNK = _IBLK // _CHUNK


def _inp_body(x_ref, w_ref, b_ref, o_ref):
    h = (
        jnp.dot(x_ref[...], w_ref[...], preferred_element_type=jnp.float32)
        + b_ref[...]
    )
    lane = lax.broadcasted_iota(jnp.int32, (_BLK, _DE - _D), 1)
    o_ref[:, : _D] = h
    o_ref[:, _D :] = jnp.where(lane == 0, 1.0, 0.0)


def _input_linear(x, W_inp, b_inp):
    return pl.pallas_call(
        _inp_body,
        grid=(_N // _BLK,),
        in_specs=[
            pl.BlockSpec((_BLK, _D), lambda i: (i, 0)),
            pl.BlockSpec((_D, _D), lambda i: (0, 0)),
            pl.BlockSpec((1, _D), lambda i: (0, 0)),
        ],
        out_specs=pl.BlockSpec((_BLK, _DE), lambda i: (i, 0)),
        out_shape=jax.ShapeDtypeStruct((_N, _DE), jnp.float32),
    )(x, W_inp, b_inp.reshape(1, _D))


def _sc_body(h_hbm, s0_hbm, d0_hbm, s1_hbm, d1_hbm,
             agg0_hbm, agg1_hbm,
             src_all, dst_all, rows_a, rows_b, agg_sh,
             sem_ga, sem_gb, sem_sa, sem_sb):
    cid = lax.axis_index("c")
    tid = lax.axis_index("s")
    base_row = tid * _RPT

    zeros16 = jnp.zeros((16,), jnp.float32)

    def _zero_rows(i, _):
        for k in range(_DE // 16):
            rows_a[i, pl.ds(k * 16, 16)] = zeros16
        return 0

    lax.fori_loop(0, _CHUNK, _zero_rows, 0)

    for k in range(_RPT // _CHUNK):
        pltpu.sync_copy(rows_a, agg_sh.at[pl.ds(base_row + k * _CHUNK, _CHUNK)])
    plsc.subcore_barrier()

    def _process(src_hbm, dst_hbm):
        def _src(c):
            return src_all.at[pl.ds(c * _CHUNK, _CHUNK)]

        def _dst(c):
            return dst_all.at[pl.ds(c * _CHUNK, _CHUNK)]

        def _start_g(c, rows, sem):
            pltpu.async_copy(h_hbm.at[_src(c)], rows, sem)

        def _wait_g(c, rows, sem):
            pltpu.make_async_copy(h_hbm.at[_src(c)], rows, sem).wait()

        def _start_s(c, rows, sem):
            pltpu.async_copy(rows, agg_sh.at[_dst(c)], sem, add=True)

        def _wait_s(c, rows, sem):
            pltpu.make_async_copy(rows, agg_sh.at[_dst(c)], sem).wait()

        def _block(b, _):
            off = tid * _EPT + b * _IBLK
            pltpu.sync_copy(src_hbm.at[pl.ds(off, _IBLK)], src_all)
            pltpu.sync_copy(dst_hbm.at[pl.ds(off, _IBLK)], dst_all)

            _start_g(0, rows_a, sem_ga)
            _start_g(1, rows_b, sem_gb)

            def body(j, _):
                c0 = 2 * j
                _wait_g(c0, rows_a, sem_ga)
                _start_s(c0, rows_a, sem_sa)
                _wait_g(c0 + 1, rows_b, sem_gb)
                _start_s(c0 + 1, rows_b, sem_sb)
                _wait_s(c0, rows_a, sem_sa)

                @pl.when(c0 + 2 < _BCHUNK)
                def _():
                    _start_g(c0 + 2, rows_a, sem_ga)

                _wait_s(c0 + 1, rows_b, sem_sb)

                @pl.when(c0 + 3 < _BCHUNK)
                def _():
                    _start_g(c0 + 3, rows_b, sem_gb)

                return 0

            lax.fori_loop(0, _BCHUNK // 2, body, 0)
            if _BCHUNK % 2 == 1:
                c = _BCHUNK - 1
                _wait_g(c, rows_a, sem_ga)
                _start_s(c, rows_a, sem_sa)
                _wait_s(c, rows_a, sem_sa)
            return 0

        lax.fori_loop(0, _EPT // _IBLK, _block, 0)

    @pl.when(cid == 0)
    def _():
        _process(s0_hbm, d0_hbm)

    @pl.when(cid == 1)
    def _():
        _process(s1_hbm, d1_hbm)

    plsc.subcore_barrier()

    def _writeback(agg_hbm):
        for k in range(_RPT // _CHUNK):
            sl = pl.ds(base_row + k * _CHUNK, _CHUNK)
            pltpu.sync_copy(agg_sh.at[sl], rows_a)
            pltpu.sync_copy(rows_a, agg_hbm.at[sl])

    @pl.when(cid == 0)
    def _():
        _writeback(agg0_hbm)

    @pl.when(cid == 1)
    def _():
        _writeback(agg1_hbm)


@functools.cache
def _make_sc_segments():
    return functools.partial(
        pl.kernel,
        out_type=(
            jax.ShapeDtypeStruct((_NPAD, _DE), jnp.float32),
            jax.ShapeDtypeStruct((_NPAD, _DE), jnp.float32),
        ),
        mesh=plsc.VectorSubcoreMesh(core_axis_name="c", subcore_axis_name="s"),
        compiler_params=pltpu.CompilerParams(use_tc_tiling_on_sc=False),
        scratch_types=[
            pltpu.VMEM((_IBLK,), jnp.int32),
            pltpu.VMEM((_IBLK,), jnp.int32),
            pltpu.VMEM((_CHUNK, _DE), jnp.float32),
            pltpu.VMEM((_CHUNK, _DE), jnp.float32),
            pltpu.VMEM_SHARED((_NPAD, _DE), jnp.float32),
            pltpu.SemaphoreType.DMA,
            pltpu.SemaphoreType.DMA,
            pltpu.SemaphoreType.DMA,
            pltpu.SemaphoreType.DMA,
        ],
    )(_sc_body)


def _head_body(agg0_ref, agg1_ref, basis_ref, wc_ref,
               hb_ref, w1_ref, b1_ref, wo_ref, bo_ref, o_ref):
    b0 = basis_ref[0]
    b1m = basis_ref[1]
    wr0 = wc_ref[0, 0] * b0 + wc_ref[0, 1] * b1m
    wr1 = wc_ref[1, 0] * b0 + wc_ref[1, 1] * b1m
    d0 = jnp.maximum(agg0_ref[:, _D : _D + 1], 1.0)
    d1 = jnp.maximum(agg1_ref[:, _D : _D + 1], 1.0)
    hc = (
        jnp.dot(agg0_ref[:, : _D] / d0, wr0, preferred_element_type=jnp.float32)
        + jnp.dot(agg1_ref[:, : _D] / d1, wr1, preferred_element_type=jnp.float32)
        + hb_ref[...]
    )
    hc = jnp.where(hc >= 0, hc, 0.01 * hc)
    h1 = jnp.dot(hc, w1_ref[...], preferred_element_type=jnp.float32) + b1_ref[...]
    h1 = jnp.where(h1 >= 0, h1, 0.01 * h1)
    lg = jnp.dot(h1, wo_ref[...], preferred_element_type=jnp.float32) + bo_ref[...]
    m = jnp.max(lg, axis=-1, keepdims=True)
    e = jnp.exp(lg - m)
    o_ref[...] = e / jnp.sum(e, axis=-1, keepdims=True)


def _head(agg0, agg1, basis, w_comp, h_bias, W1, b1, Wout, bout):
    full = lambda shape: pl.BlockSpec(shape, lambda i: tuple(0 for _ in shape))
    return pl.pallas_call(
        _head_body,
        grid=(_N // _BLK,),
        in_specs=[
            pl.BlockSpec((_BLK, _DE), lambda i: (i, 0)),
            pl.BlockSpec((_BLK, _DE), lambda i: (i, 0)),
            full((2, _D, _D)),
            full((2, 2)),
            full((1, _D)),
            full((_D, 64)),
            full((1, 64)),
            full((64, 16)),
            full((1, 16)),
        ],
        out_specs=pl.BlockSpec((_BLK, 16), lambda i: (i, 0)),
        out_shape=jax.ShapeDtypeStruct((_N, 16), jnp.float32),
    )(agg0, agg1, basis, w_comp, h_bias.reshape(1, _D),
      W1, b1.reshape(1, 64), Wout, bout.reshape(1, 16))


def kernel(x, edge_index_rel0, edge_index_rel1, W_inp, b_inp, basis, w_comp,
           h_bias, W1, b1, Wout, bout):
    h_ext = _input_linear(x, W_inp, b_inp)
    agg0, agg1 = _make_sc_segments()(
        h_ext,
        edge_index_rel0[0], edge_index_rel0[1],
        edge_index_rel1[0], edge_index_rel1[1],
    )
    return _head(agg0, agg1, basis, w_comp, h_bias, W1, b1, Wout, bout)


# speedup vs baseline: 1.1019x; 1.1019x over previous
import functools

import jax
import jax.numpy as jnp
from jax import lax
from jax.experimental import pallas as pl
from jax.experimental.pallas import tpu as pltpu
from jax.experimental.pallas import tpu_sc as plsc

_N = 10000
_D = 128
_DE = 144
_E = 160000
_BLK = 80
_CHUNK = 80
_TILES = 16
_NPAD = 10240
_RPT = _NPAD // _TILES
_EPT = _E // _TILES
_NCHUNK = _EPT // _CHUNK
_IBLK = 2000
_BCHUNK = _IBLK // _CHUNK


def _inp_body(x_ref, w_ref, b_ref, o_ref):
    h = (
        jnp.dot(x_ref[...], w_ref[...], preferred_element_type=jnp.float32)
        + b_ref[...]
    )
    lane = lax.broadcasted_iota(jnp.int32, (_BLK, _DE - _D), 1)
    o_ref[:, : _D] = h
    o_ref[:, _D :] = jnp.where(lane == 0, 1.0, 0.0)


def _input_linear(x, W_inp, b_inp):
    return pl.pallas_call(
        _inp_body,
        grid=(_N // _BLK,),
        in_specs=[
            pl.BlockSpec((_BLK, _D), lambda i: (i, 0)),
            pl.BlockSpec((_D, _D), lambda i: (0, 0)),
            pl.BlockSpec((1, _D), lambda i: (0, 0)),
        ],
        out_specs=pl.BlockSpec((_BLK, _DE), lambda i: (i, 0)),
        out_shape=jax.ShapeDtypeStruct((_N, _DE), jnp.float32),
    )(x, W_inp, b_inp.reshape(1, _D))


def _sc_body(h_hbm, s0_hbm, d0_hbm, s1_hbm, d1_hbm,
             agg0_hbm, agg1_hbm,
             src_all, dst_all, rows_a, rows_b, agg_sh,
             sem_ga, sem_gb, sem_sa, sem_sb):
    cid = lax.axis_index("c")
    tid = lax.axis_index("s")
    base_row = tid * _RPT

    zeros16 = jnp.zeros((16,), jnp.float32)

    def _zero_rows(i, _):
        for k in range(_DE // 16):
            rows_a[i, pl.ds(k * 16, 16)] = zeros16
        return 0

    lax.fori_loop(0, _CHUNK, _zero_rows, 0)

    for k in range(_RPT // _CHUNK):
        pltpu.sync_copy(rows_a, agg_sh.at[pl.ds(base_row + k * _CHUNK, _CHUNK)])
    plsc.subcore_barrier()

    def _process(src_hbm, dst_hbm):
        def _src(c):
            return src_all.at[pl.ds(c * _CHUNK, _CHUNK)]

        def _dst(c):
            return dst_all.at[pl.ds(c * _CHUNK, _CHUNK)]

        def _start_g(c, rows, sem):
            pltpu.async_copy(h_hbm.at[_src(c)], rows, sem)

        def _wait_g(c, rows, sem):
            pltpu.make_async_copy(h_hbm.at[_src(c)], rows, sem).wait()

        def _start_s(c, rows, sem):
            pltpu.async_copy(rows, agg_sh.at[_dst(c)], sem, add=True)

        def _wait_s(c, rows, sem):
            pltpu.make_async_copy(rows, agg_sh.at[_dst(c)], sem).wait()

        def _block(b, _):
            off = tid * _EPT + b * _IBLK
            pltpu.sync_copy(src_hbm.at[pl.ds(off, _IBLK)], src_all)
            pltpu.sync_copy(dst_hbm.at[pl.ds(off, _IBLK)], dst_all)

            _start_g(0, rows_a, sem_ga)

            def _finish(c, rows, sem):
                _wait_g(c, rows, sem)
                pltpu.sync_copy(rows, agg_sh.at[_dst(c)], add=True)

            def body(j, _):
                c0 = 2 * j
                _start_g(c0 + 1, rows_b, sem_gb)
                _finish(c0, rows_a, sem_ga)

                @pl.when(c0 + 2 < _BCHUNK)
                def _():
                    _start_g(c0 + 2, rows_a, sem_ga)

                _finish(c0 + 1, rows_b, sem_gb)
                return 0

            lax.fori_loop(0, _BCHUNK // 2, body, 0)
            if _BCHUNK % 2 == 1:
                _finish(_BCHUNK - 1, rows_a, sem_ga)
            return 0

        lax.fori_loop(0, _EPT // _IBLK, _block, 0)

    @pl.when(cid == 0)
    def _():
        _process(s0_hbm, d0_hbm)

    @pl.when(cid == 1)
    def _():
        _process(s1_hbm, d1_hbm)

    plsc.subcore_barrier()

    def _writeback(agg_hbm):
        for k in range(_RPT // _CHUNK):
            sl = pl.ds(base_row + k * _CHUNK, _CHUNK)
            pltpu.sync_copy(agg_sh.at[sl], rows_a)
            pltpu.sync_copy(rows_a, agg_hbm.at[sl])

    @pl.when(cid == 0)
    def _():
        _writeback(agg0_hbm)

    @pl.when(cid == 1)
    def _():
        _writeback(agg1_hbm)


@functools.cache
def _make_sc_segments():
    return functools.partial(
        pl.kernel,
        out_type=(
            jax.ShapeDtypeStruct((_NPAD, _DE), jnp.float32),
            jax.ShapeDtypeStruct((_NPAD, _DE), jnp.float32),
        ),
        mesh=plsc.VectorSubcoreMesh(core_axis_name="c", subcore_axis_name="s"),
        compiler_params=pltpu.CompilerParams(use_tc_tiling_on_sc=False),
        scratch_types=[
            pltpu.VMEM((_IBLK,), jnp.int32),
            pltpu.VMEM((_IBLK,), jnp.int32),
            pltpu.VMEM((_CHUNK, _DE), jnp.float32),
            pltpu.VMEM((_CHUNK, _DE), jnp.float32),
            pltpu.VMEM_SHARED((_NPAD, _DE), jnp.float32),
            pltpu.SemaphoreType.DMA,
            pltpu.SemaphoreType.DMA,
            pltpu.SemaphoreType.DMA,
            pltpu.SemaphoreType.DMA,
        ],
    )(_sc_body)


def _head_body(agg0_ref, agg1_ref, basis_ref, wc_ref,
               hb_ref, w1_ref, b1_ref, wo_ref, bo_ref, o_ref):
    b0 = basis_ref[0]
    b1m = basis_ref[1]
    wr0 = wc_ref[0, 0] * b0 + wc_ref[0, 1] * b1m
    wr1 = wc_ref[1, 0] * b0 + wc_ref[1, 1] * b1m
    d0 = jnp.maximum(agg0_ref[:, _D : _D + 1], 1.0)
    d1 = jnp.maximum(agg1_ref[:, _D : _D + 1], 1.0)
    hc = (
        jnp.dot(agg0_ref[:, : _D] / d0, wr0, preferred_element_type=jnp.float32)
        + jnp.dot(agg1_ref[:, : _D] / d1, wr1, preferred_element_type=jnp.float32)
        + hb_ref[...]
    )
    hc = jnp.where(hc >= 0, hc, 0.01 * hc)
    h1 = jnp.dot(hc, w1_ref[...], preferred_element_type=jnp.float32) + b1_ref[...]
    h1 = jnp.where(h1 >= 0, h1, 0.01 * h1)
    lg = jnp.dot(h1, wo_ref[...], preferred_element_type=jnp.float32) + bo_ref[...]
    m = jnp.max(lg, axis=-1, keepdims=True)
    e = jnp.exp(lg - m)
    o_ref[...] = e / jnp.sum(e, axis=-1, keepdims=True)


def _head(agg0, agg1, basis, w_comp, h_bias, W1, b1, Wout, bout):
    full = lambda shape: pl.BlockSpec(shape, lambda i: tuple(0 for _ in shape))
    return pl.pallas_call(
        _head_body,
        grid=(_N // _BLK,),
        in_specs=[
            pl.BlockSpec((_BLK, _DE), lambda i: (i, 0)),
            pl.BlockSpec((_BLK, _DE), lambda i: (i, 0)),
            full((2, _D, _D)),
            full((2, 2)),
            full((1, _D)),
            full((_D, 64)),
            full((1, 64)),
            full((64, 16)),
            full((1, 16)),
        ],
        out_specs=pl.BlockSpec((_BLK, 16), lambda i: (i, 0)),
        out_shape=jax.ShapeDtypeStruct((_N, 16), jnp.float32),
    )(agg0, agg1, basis, w_comp, h_bias.reshape(1, _D),
      W1, b1.reshape(1, 64), Wout, bout.reshape(1, 16))


def kernel(x, edge_index_rel0, edge_index_rel1, W_inp, b_inp, basis, w_comp,
           h_bias, W1, b1, Wout, bout):
    h_ext = _input_linear(x, W_inp, b_inp)
    agg0, agg1 = _make_sc_segments()(
        h_ext,
        edge_index_rel0[0], edge_index_rel0[1],
        edge_index_rel1[0], edge_index_rel1[1],
    )
    return _head(agg0, agg1, basis, w_comp, h_bias, W1, b1, Wout, bout)


# speedup vs baseline: 1.6697x; 1.5153x over previous
import functools

import jax
import jax.numpy as jnp
from jax import lax
from jax.experimental import pallas as pl
from jax.experimental.pallas import tpu as pltpu
from jax.experimental.pallas import tpu_sc as plsc

_N = 10000
_D = 128
_DE = 144
_E = 160000
_BLK = 400
_CHUNK = 80
_TILES = 16
_NPAD = 10240
_RPT = _NPAD // _TILES
_EPT = _E // _TILES
_NCHUNK = _EPT // _CHUNK
_IBLK = 2000
_BCHUNK = _IBLK // _CHUNK


def _inp_body(x_ref, w_ref, b_ref, o_ref):
    h = (
        jnp.dot(x_ref[...], w_ref[...], preferred_element_type=jnp.float32)
        + b_ref[...]
    )
    lane = lax.broadcasted_iota(jnp.int32, (_BLK, _DE - _D), 1)
    o_ref[:, : _D] = h
    o_ref[:, _D :] = jnp.where(lane == 0, 1.0, 0.0)


def _input_linear(x, W_inp, b_inp):
    return pl.pallas_call(
        _inp_body,
        grid=(_N // _BLK,),
        in_specs=[
            pl.BlockSpec((_BLK, _D), lambda i: (i, 0)),
            pl.BlockSpec((_D, _D), lambda i: (0, 0)),
            pl.BlockSpec((1, _D), lambda i: (0, 0)),
        ],
        out_specs=pl.BlockSpec((_BLK, _DE), lambda i: (i, 0)),
        out_shape=jax.ShapeDtypeStruct((_N, _DE), jnp.float32),
    )(x, W_inp, b_inp.reshape(1, _D))


def _sc_body(h_hbm, s0_hbm, d0_hbm, s1_hbm, d1_hbm,
             agg0_hbm, agg1_hbm,
             src_all, dst_all, rows_a, rows_b, agg_sh,
             sem_ga, sem_gb, sem_sa, sem_sb):
    cid = lax.axis_index("c")
    tid = lax.axis_index("s")
    base_row = tid * _RPT

    zeros16 = jnp.zeros((16,), jnp.float32)

    def _zero_rows(i, _):
        for k in range(_DE // 16):
            rows_a[i, pl.ds(k * 16, 16)] = zeros16
        return 0

    lax.fori_loop(0, _CHUNK, _zero_rows, 0)

    for k in range(_RPT // _CHUNK):
        pltpu.sync_copy(rows_a, agg_sh.at[pl.ds(base_row + k * _CHUNK, _CHUNK)])
    plsc.subcore_barrier()

    def _process(src_hbm, dst_hbm):
        def _src(c):
            return src_all.at[pl.ds(c * _CHUNK, _CHUNK)]

        def _dst(c):
            return dst_all.at[pl.ds(c * _CHUNK, _CHUNK)]

        def _start_g(c, rows, sem):
            pltpu.async_copy(h_hbm.at[_src(c)], rows, sem)

        def _wait_g(c, rows, sem):
            pltpu.make_async_copy(h_hbm.at[_src(c)], rows, sem).wait()

        def _start_s(c, rows, sem):
            pltpu.async_copy(rows, agg_sh.at[_dst(c)], sem, add=True)

        def _wait_s(c, rows, sem):
            pltpu.make_async_copy(rows, agg_sh.at[_dst(c)], sem).wait()

        def _block(b, _):
            off = tid * _EPT + b * _IBLK
            pltpu.sync_copy(src_hbm.at[pl.ds(off, _IBLK)], src_all)
            pltpu.sync_copy(dst_hbm.at[pl.ds(off, _IBLK)], dst_all)

            _start_g(0, rows_a, sem_ga)

            def _finish(c, rows, sem):
                _wait_g(c, rows, sem)
                pltpu.sync_copy(rows, agg_sh.at[_dst(c)], add=True)

            def body(j, _):
                c0 = 2 * j
                _start_g(c0 + 1, rows_b, sem_gb)
                _finish(c0, rows_a, sem_ga)

                @pl.when(c0 + 2 < _BCHUNK)
                def _():
                    _start_g(c0 + 2, rows_a, sem_ga)

                _finish(c0 + 1, rows_b, sem_gb)
                return 0

            lax.fori_loop(0, _BCHUNK // 2, body, 0)
            if _BCHUNK % 2 == 1:
                _finish(_BCHUNK - 1, rows_a, sem_ga)
            return 0

        lax.fori_loop(0, _EPT // _IBLK, _block, 0)

    @pl.when(cid == 0)
    def _():
        _process(s0_hbm, d0_hbm)

    @pl.when(cid == 1)
    def _():
        _process(s1_hbm, d1_hbm)

    plsc.subcore_barrier()

    def _writeback(agg_hbm):
        sl = pl.ds(base_row, _RPT)
        pltpu.sync_copy(agg_sh.at[sl], agg_hbm.at[sl])

    @pl.when(cid == 0)
    def _():
        _writeback(agg0_hbm)

    @pl.when(cid == 1)
    def _():
        _writeback(agg1_hbm)


@functools.cache
def _make_sc_segments():
    return functools.partial(
        pl.kernel,
        out_type=(
            jax.ShapeDtypeStruct((_NPAD, _DE), jnp.float32),
            jax.ShapeDtypeStruct((_NPAD, _DE), jnp.float32),
        ),
        mesh=plsc.VectorSubcoreMesh(core_axis_name="c", subcore_axis_name="s"),
        compiler_params=pltpu.CompilerParams(use_tc_tiling_on_sc=False),
        scratch_types=[
            pltpu.VMEM((_IBLK,), jnp.int32),
            pltpu.VMEM((_IBLK,), jnp.int32),
            pltpu.VMEM((_CHUNK, _DE), jnp.float32),
            pltpu.VMEM((_CHUNK, _DE), jnp.float32),
            pltpu.VMEM_SHARED((_NPAD, _DE), jnp.float32),
            pltpu.SemaphoreType.DMA,
            pltpu.SemaphoreType.DMA,
            pltpu.SemaphoreType.DMA,
            pltpu.SemaphoreType.DMA,
        ],
    )(_sc_body)


def _head_body(agg0_ref, agg1_ref, basis_ref, wc_ref,
               hb_ref, w1_ref, b1_ref, wo_ref, bo_ref, o_ref):
    b0 = basis_ref[0]
    b1m = basis_ref[1]
    wr0 = wc_ref[0, 0] * b0 + wc_ref[0, 1] * b1m
    wr1 = wc_ref[1, 0] * b0 + wc_ref[1, 1] * b1m
    d0 = jnp.maximum(agg0_ref[:, _D : _D + 1], 1.0)
    d1 = jnp.maximum(agg1_ref[:, _D : _D + 1], 1.0)
    hc = (
        jnp.dot(agg0_ref[:, : _D] / d0, wr0, preferred_element_type=jnp.float32)
        + jnp.dot(agg1_ref[:, : _D] / d1, wr1, preferred_element_type=jnp.float32)
        + hb_ref[...]
    )
    hc = jnp.where(hc >= 0, hc, 0.01 * hc)
    h1 = jnp.dot(hc, w1_ref[...], preferred_element_type=jnp.float32) + b1_ref[...]
    h1 = jnp.where(h1 >= 0, h1, 0.01 * h1)
    lg = jnp.dot(h1, wo_ref[...], preferred_element_type=jnp.float32) + bo_ref[...]
    m = jnp.max(lg, axis=-1, keepdims=True)
    e = jnp.exp(lg - m)
    o_ref[...] = e / jnp.sum(e, axis=-1, keepdims=True)


def _head(agg0, agg1, basis, w_comp, h_bias, W1, b1, Wout, bout):
    full = lambda shape: pl.BlockSpec(shape, lambda i: tuple(0 for _ in shape))
    return pl.pallas_call(
        _head_body,
        grid=(_N // _BLK,),
        in_specs=[
            pl.BlockSpec((_BLK, _DE), lambda i: (i, 0)),
            pl.BlockSpec((_BLK, _DE), lambda i: (i, 0)),
            full((2, _D, _D)),
            full((2, 2)),
            full((1, _D)),
            full((_D, 64)),
            full((1, 64)),
            full((64, 16)),
            full((1, 16)),
        ],
        out_specs=pl.BlockSpec((_BLK, 16), lambda i: (i, 0)),
        out_shape=jax.ShapeDtypeStruct((_N, 16), jnp.float32),
    )(agg0, agg1, basis, w_comp, h_bias.reshape(1, _D),
      W1, b1.reshape(1, 64), Wout, bout.reshape(1, 16))


def kernel(x, edge_index_rel0, edge_index_rel1, W_inp, b_inp, basis, w_comp,
           h_bias, W1, b1, Wout, bout):
    h_ext = _input_linear(x, W_inp, b_inp)
    agg0, agg1 = _make_sc_segments()(
        h_ext,
        edge_index_rel0[0], edge_index_rel0[1],
        edge_index_rel1[0], edge_index_rel1[1],
    )
    return _head(agg0, agg1, basis, w_comp, h_bias, W1, b1, Wout, bout)
